# baseline (speedup 1.0000x reference)
import jax
import jax.numpy as jnp
from jax import lax
from jax.experimental import pallas as pl
from jax.experimental.pallas import tpu as pltpu
from jax.experimental.pallas import tpu_sc as plsc

_VOCAB = 8192
_D = 768
_ALPHA = 0.1
_MASK_TOK = 2

_NC, _NS, _L = 2, 16, 16
_NW = _NC * _NS

_VSC = 2048
_V0 = _VOCAB - _VSC
_NWACT = 16
_VPW = _VSC // _NWACT
_KV = _VPW // _L
_TILE = 2048

_mesh = plsc.VectorSubcoreMesh(core_axis_name="c", subcore_axis_name="s")


def _sc_matvec_body(emb_hbm, W_hbm, b_hbm, val_hbm, idx_hbm,
                    v_vmem, w_vmem, b_vmem, bv_vmem, bi_vmem):
    wid = lax.axis_index("s") * _NC + lax.axis_index("c")

    @pl.when(wid < _NWACT)
    def _():
        j0 = _V0 + wid * _VPW

        pltpu.sync_copy(emb_hbm.at[_MASK_TOK], v_vmem)
        pltpu.sync_copy(b_hbm.at[pl.ds(j0, _VPW)], b_vmem)
        pltpu.sync_copy(W_hbm.at[:, pl.ds(j0, _VPW)], w_vmem)

        acc = tuple(jnp.zeros((_L,), jnp.float32) for _ in range(_KV))

        def gbody(g, a):
            vchunk = v_vmem[pl.ds(g * _L, _L)]
            for l in range(_L):
                vd = vchunk[l]
                d = g * _L + l
                a = tuple(
                    ak + vd * w_vmem[d, pl.ds(k * _L, _L)]
                    for k, ak in enumerate(a))
            return a

        acc = lax.fori_loop(0, _D // _L, gbody, acc)

        bestv = acc[0] + b_vmem[pl.ds(0, _L)]
        besti = j0 + lax.iota(jnp.int32, _L)
        for k in range(1, _KV):
            s_k = acc[k] + b_vmem[pl.ds(k * _L, _L)]
            i_k = j0 + k * _L + lax.iota(jnp.int32, _L)
            upd = s_k > bestv
            bestv = jnp.where(upd, s_k, bestv)
            besti = jnp.where(upd, i_k, besti)
        bv_vmem[...] = bestv
        bi_vmem[...] = besti
        pltpu.sync_copy(bv_vmem, val_hbm.at[pl.ds(wid * _L, _L)])
        pltpu.sync_copy(bi_vmem, idx_hbm.at[pl.ds(wid * _L, _L)])


def _tc_matvec_body(emb_ref, W_ref, b_ref, m_ref, i_ref, bestv_ref, besti_ref):
    j = pl.program_id(0)
    nj = pl.num_programs(0)

    v = emb_ref[_MASK_TOK : _MASK_TOK + 1, :]
    s = (
        jax.lax.dot_general(
            v, W_ref[...], (((1,), (0,)), ((), ())),
            preferred_element_type=jnp.float32,
        )
        + b_ref[...]
    )

    m = jnp.max(s)
    idx = jax.lax.broadcasted_iota(jnp.int32, s.shape, 1)
    a = jnp.min(jnp.where(s == m, idx, _TILE))

    @pl.when(j == 0)
    def _():
        bestv_ref[0] = m
        besti_ref[0] = a

    @pl.when((j > 0) & (m > bestv_ref[0]))
    def _():
        bestv_ref[0] = m
        besti_ref[0] = j * _TILE + a

    @pl.when(j == nj - 1)
    def _():
        m_ref[...] = jnp.full((1, 128), bestv_ref[0])
        i_ref[...] = jnp.full((1, 128), besti_ref[0])


def _combine_body(m_ref, i_ref, val_ref, idx_ref, X_ref, rand_ref, out_ref):
    val = val_ref[...]
    idx = idx_ref[...]
    m2 = jnp.max(val)
    i2 = jnp.min(jnp.where(val == m2, idx, _VOCAB))
    m_tc = m_ref[0, 0]
    i_tc = i_ref[0, 0]
    p = jnp.where(m_tc >= m2, i_tc, i2).astype(jnp.float32)
    X = X_ref[...]
    cond = (X == jnp.float32(_MASK_TOK)) & (rand_ref[...] < jnp.float32(_ALPHA))
    out_ref[...] = jnp.where(cond, p, X)


def kernel(X, rand_vals, emb, W, b):
    Bsz, L = X.shape
    b2 = b.reshape(1, _VOCAB)

    val, idx = pl.kernel(
        _sc_matvec_body,
        out_type=[
            jax.ShapeDtypeStruct((_NWACT * _L,), jnp.float32),
            jax.ShapeDtypeStruct((_NWACT * _L,), jnp.int32),
        ],
        mesh=_mesh,
        scratch_types=[
            pltpu.VMEM((_D,), jnp.float32),
            pltpu.VMEM((_D, _VPW), jnp.float32),
            pltpu.VMEM((_VPW,), jnp.float32),
            pltpu.VMEM((_L,), jnp.float32),
            pltpu.VMEM((_L,), jnp.int32),
        ],
    )(emb, W, b)

    m_row, i_row = pl.pallas_call(
        _tc_matvec_body,
        grid=(_V0 // _TILE,),
        in_specs=[
            pl.BlockSpec((8, _D), lambda j: (0, 0)),
            pl.BlockSpec((_D, _TILE), lambda j: (0, j)),
            pl.BlockSpec((1, _TILE), lambda j: (0, j)),
        ],
        out_specs=[
            pl.BlockSpec((1, 128), lambda j: (0, 0)),
            pl.BlockSpec((1, 128), lambda j: (0, 0)),
        ],
        out_shape=[
            jax.ShapeDtypeStruct((1, 128), jnp.float32),
            jax.ShapeDtypeStruct((1, 128), jnp.int32),
        ],
        scratch_shapes=[
            pltpu.SMEM((1,), jnp.float32),
            pltpu.SMEM((1,), jnp.int32),
        ],
    )(emb, W, b2)

    out = pl.pallas_call(
        _combine_body,
        in_specs=[
            pl.BlockSpec((1, 128), lambda: (0, 0)),
            pl.BlockSpec((1, 128), lambda: (0, 0)),
            pl.BlockSpec((2, 128), lambda: (0, 0)),
            pl.BlockSpec((2, 128), lambda: (0, 0)),
            pl.BlockSpec((Bsz, L), lambda: (0, 0)),
            pl.BlockSpec((Bsz, L), lambda: (0, 0)),
        ],
        out_specs=pl.BlockSpec((Bsz, L), lambda: (0, 0)),
        out_shape=jax.ShapeDtypeStruct((Bsz, L), X.dtype),
    )(m_row, i_row, val.reshape(2, 128), idx.reshape(2, 128), X, rand_vals)
    return out


# speedup vs baseline: 1.0268x; 1.0268x over previous
import jax
import jax.numpy as jnp
from jax import lax
from jax.experimental import pallas as pl
from jax.experimental.pallas import tpu as pltpu
from jax.experimental.pallas import tpu_sc as plsc

_VOCAB = 8192
_D = 768
_ALPHA = 0.1
_MASK_TOK = 2

_NC, _NS, _L = 2, 16, 16
_NW = _NC * _NS

_VSC = 4096
_V0 = _VOCAB - _VSC
_NWACT = 32
_VPW = _VSC // _NWACT
_KV = _VPW // _L
_DCH = 192
_NCH = _D // _DCH
_TILE = 2048

_mesh = plsc.VectorSubcoreMesh(core_axis_name="c", subcore_axis_name="s")


def _sc_matvec_body(emb_hbm, W_hbm, b_hbm, val_hbm, idx_hbm,
                    v_vmem, w0, w1, b_vmem, bv_vmem, bi_vmem, sem0, sem1):
    wid = lax.axis_index("s") * _NC + lax.axis_index("c")
    j0 = _V0 + wid * _VPW

    bufs = (w0, w1)
    sems = (sem0, sem1)
    cps = [None, None]
    cps[0] = pltpu.async_copy(
        W_hbm.at[pl.ds(0, _DCH), pl.ds(j0, _VPW)], w0, sem0)
    pltpu.sync_copy(emb_hbm.at[_MASK_TOK], v_vmem)
    pltpu.sync_copy(b_hbm.at[pl.ds(j0, _VPW)], b_vmem)

    acc = tuple(jnp.zeros((_L,), jnp.float32) for _ in range(_KV))
    for c in range(_NCH):
        cps[c % 2].wait()
        if c + 1 < _NCH:
            cps[(c + 1) % 2] = pltpu.async_copy(
                W_hbm.at[pl.ds((c + 1) * _DCH, _DCH), pl.ds(j0, _VPW)],
                bufs[(c + 1) % 2], sems[(c + 1) % 2])
        wbuf = bufs[c % 2]

        def gbody(g, a, _c=c, _w=wbuf):
            vchunk = v_vmem[pl.ds(_c * _DCH + g * _L, _L)]
            for l in range(_L):
                vd = vchunk[l]
                d = g * _L + l
                a = tuple(
                    ak + vd * _w[d, pl.ds(k * _L, _L)]
                    for k, ak in enumerate(a))
            return a

        acc = lax.fori_loop(0, _DCH // _L, gbody, acc)

    bestv = acc[0] + b_vmem[pl.ds(0, _L)]
    besti = j0 + lax.iota(jnp.int32, _L)
    for k in range(1, _KV):
        s_k = acc[k] + b_vmem[pl.ds(k * _L, _L)]
        i_k = j0 + k * _L + lax.iota(jnp.int32, _L)
        upd = s_k > bestv
        bestv = jnp.where(upd, s_k, bestv)
        besti = jnp.where(upd, i_k, besti)
    bv_vmem[...] = bestv
    bi_vmem[...] = besti
    pltpu.sync_copy(bv_vmem, val_hbm.at[pl.ds(wid * _L, _L)])
    pltpu.sync_copy(bi_vmem, idx_hbm.at[pl.ds(wid * _L, _L)])


def _tc_matvec_body(emb_ref, W_ref, b_ref, m_ref, i_ref, bestv_ref, besti_ref):
    j = pl.program_id(0)
    nj = pl.num_programs(0)

    v = emb_ref[_MASK_TOK : _MASK_TOK + 1, :]
    s = (
        jax.lax.dot_general(
            v, W_ref[...], (((1,), (0,)), ((), ())),
            preferred_element_type=jnp.float32,
        )
        + b_ref[...]
    )

    m = jnp.max(s)
    idx = jax.lax.broadcasted_iota(jnp.int32, s.shape, 1)
    a = jnp.min(jnp.where(s == m, idx, _TILE))

    @pl.when(j == 0)
    def _():
        bestv_ref[0] = m
        besti_ref[0] = a

    @pl.when((j > 0) & (m > bestv_ref[0]))
    def _():
        bestv_ref[0] = m
        besti_ref[0] = j * _TILE + a

    @pl.when(j == nj - 1)
    def _():
        m_ref[...] = jnp.full((1, 128), bestv_ref[0])
        i_ref[...] = jnp.full((1, 128), besti_ref[0])


def _combine_body(m_ref, i_ref, val_ref, idx_ref, X_ref, rand_ref, out_ref):
    val = val_ref[...]
    idx = idx_ref[...]
    m2 = jnp.max(val)
    i2 = jnp.min(jnp.where(val == m2, idx, _VOCAB))
    m_tc = m_ref[0, 0]
    i_tc = i_ref[0, 0]
    p = jnp.where(m_tc >= m2, i_tc, i2).astype(jnp.float32)
    X = X_ref[...]
    cond = (X == jnp.float32(_MASK_TOK)) & (rand_ref[...] < jnp.float32(_ALPHA))
    out_ref[...] = jnp.where(cond, p, X)


def kernel(X, rand_vals, emb, W, b):
    Bsz, L = X.shape
    b2 = b.reshape(1, _VOCAB)

    val, idx = pl.kernel(
        _sc_matvec_body,
        out_type=[
            jax.ShapeDtypeStruct((_NWACT * _L,), jnp.float32),
            jax.ShapeDtypeStruct((_NWACT * _L,), jnp.int32),
        ],
        mesh=_mesh,
        scratch_types=[
            pltpu.VMEM((_D,), jnp.float32),
            pltpu.VMEM((_DCH, _VPW), jnp.float32),
            pltpu.VMEM((_DCH, _VPW), jnp.float32),
            pltpu.VMEM((_VPW,), jnp.float32),
            pltpu.VMEM((_L,), jnp.float32),
            pltpu.VMEM((_L,), jnp.int32),
            pltpu.SemaphoreType.DMA,
            pltpu.SemaphoreType.DMA,
        ],
    )(emb, W, b)

    m_row, i_row = pl.pallas_call(
        _tc_matvec_body,
        grid=(_V0 // _TILE,),
        in_specs=[
            pl.BlockSpec((8, _D), lambda j: (0, 0)),
            pl.BlockSpec((_D, _TILE), lambda j: (0, j)),
            pl.BlockSpec((1, _TILE), lambda j: (0, j)),
        ],
        out_specs=[
            pl.BlockSpec((1, 128), lambda j: (0, 0)),
            pl.BlockSpec((1, 128), lambda j: (0, 0)),
        ],
        out_shape=[
            jax.ShapeDtypeStruct((1, 128), jnp.float32),
            jax.ShapeDtypeStruct((1, 128), jnp.int32),
        ],
        scratch_shapes=[
            pltpu.SMEM((1,), jnp.float32),
            pltpu.SMEM((1,), jnp.int32),
        ],
    )(emb, W, b2)

    out = pl.pallas_call(
        _combine_body,
        in_specs=[
            pl.BlockSpec((1, 128), lambda: (0, 0)),
            pl.BlockSpec((1, 128), lambda: (0, 0)),
            pl.BlockSpec((4, 128), lambda: (0, 0)),
            pl.BlockSpec((4, 128), lambda: (0, 0)),
            pl.BlockSpec((Bsz, L), lambda: (0, 0)),
            pl.BlockSpec((Bsz, L), lambda: (0, 0)),
        ],
        out_specs=pl.BlockSpec((Bsz, L), lambda: (0, 0)),
        out_shape=jax.ShapeDtypeStruct((Bsz, L), X.dtype),
    )(m_row, i_row, val.reshape(4, 128), idx.reshape(4, 128), X, rand_vals)
    return out


# speedup vs baseline: 1.1674x; 1.1369x over previous
import jax
import jax.numpy as jnp
from jax import lax
from jax.experimental import pallas as pl
from jax.experimental.pallas import tpu as pltpu
from jax.experimental.pallas import tpu_sc as plsc

_VOCAB = 8192
_D = 768
_ALPHA = 0.1
_MASK_TOK = 2
_TILE = 4096

_NC, _NS, _L = 2, 16, 16
_NW = _NC * _NS
_BL = 2 * 2048
_XPW = _BL // _NW

_mesh = plsc.VectorSubcoreMesh(core_axis_name="c", subcore_axis_name="s")


def _matvec_body(emb_ref, W_ref, b_ref, p_ref, bestv_ref, besti_ref):
    j = pl.program_id(0)
    nj = pl.num_programs(0)

    v = emb_ref[_MASK_TOK : _MASK_TOK + 1, :]
    s = (
        jax.lax.dot_general(
            v, W_ref[...], (((1,), (0,)), ((), ())),
            preferred_element_type=jnp.float32,
        )
        + b_ref[...]
    )

    m = jnp.max(s)
    idx = jax.lax.broadcasted_iota(jnp.int32, s.shape, 1)
    a = jnp.min(jnp.where(s == m, idx, _TILE))

    @pl.when(j == 0)
    def _():
        bestv_ref[0] = m
        besti_ref[0] = a

    @pl.when((j > 0) & (m > bestv_ref[0]))
    def _():
        bestv_ref[0] = m
        besti_ref[0] = j * _TILE + a

    @pl.when(j == nj - 1)
    def _():
        p_ref[...] = jnp.full((1, 128), besti_ref[0].astype(jnp.float32))


def _select_body(p_hbm, x_hbm, r_hbm, out_hbm, p_v, x_v, r_v, o_v):
    wid = lax.axis_index("s") * _NC + lax.axis_index("c")
    row = wid // (2048 // _XPW)
    col = (wid % (2048 // _XPW)) * _XPW
    pltpu.sync_copy(p_hbm.at[0], p_v)
    pltpu.sync_copy(x_hbm.at[row, pl.ds(col, _XPW)], x_v)
    pltpu.sync_copy(r_hbm.at[row, pl.ds(col, _XPW)], r_v)
    pb = p_v[pl.ds(0, _L)]

    for k in range(_XPW // _L):
        xk = x_v[pl.ds(k * _L, _L)]
        rk = r_v[pl.ds(k * _L, _L)]
        cond = (xk == jnp.float32(_MASK_TOK)) & (rk < jnp.float32(_ALPHA))
        o_v[pl.ds(k * _L, _L)] = jnp.where(cond, pb, xk)
    pltpu.sync_copy(o_v, out_hbm.at[row, pl.ds(col, _XPW)])


def kernel(X, rand_vals, emb, W, b):
    b2 = b.reshape(1, _VOCAB)
    p_row = pl.pallas_call(
        _matvec_body,
        grid=(_VOCAB // _TILE,),
        in_specs=[
            pl.BlockSpec((8, _D), lambda j: (0, 0)),
            pl.BlockSpec((_D, _TILE), lambda j: (0, j)),
            pl.BlockSpec((1, _TILE), lambda j: (0, j)),
        ],
        out_specs=pl.BlockSpec((1, 128), lambda j: (0, 0)),
        out_shape=jax.ShapeDtypeStruct((1, 128), jnp.float32),
        scratch_shapes=[
            pltpu.SMEM((1,), jnp.float32),
            pltpu.SMEM((1,), jnp.int32),
        ],
    )(emb, W, b2)

    out = pl.kernel(
        _select_body,
        out_type=jax.ShapeDtypeStruct(X.shape, jnp.float32),
        mesh=_mesh,
        scratch_types=[
            pltpu.VMEM((128,), jnp.float32),
            pltpu.VMEM((_XPW,), jnp.float32),
            pltpu.VMEM((_XPW,), jnp.float32),
            pltpu.VMEM((_XPW,), jnp.float32),
        ],
    )(p_row, X, rand_vals)
    return out


# speedup vs baseline: 1.2494x; 1.0703x over previous
import jax
import jax.numpy as jnp
from jax import lax
from jax.experimental import pallas as pl
from jax.experimental.pallas import tpu as pltpu
from jax.experimental.pallas import tpu_sc as plsc

_VOCAB = 8192
_D = 768
_ALPHA = 0.1
_MASK_TOK = 2
_TILE = 4096

_NC, _NS, _L = 1, 16, 16
_NW = _NC * _NS
_BL = 2 * 2048
_XPW = _BL // _NW

_mesh = plsc.VectorSubcoreMesh(core_axis_name="c", subcore_axis_name="s", num_cores=1)


def _matvec_body(emb_ref, W_ref, b_ref, p_ref, bestv_ref, besti_ref):
    j = pl.program_id(0)
    nj = pl.num_programs(0)

    v = emb_ref[_MASK_TOK : _MASK_TOK + 1, :]
    s = (
        jax.lax.dot_general(
            v, W_ref[...], (((1,), (0,)), ((), ())),
            preferred_element_type=jnp.float32,
        )
        + b_ref[...]
    )

    m = jnp.max(s)
    idx = jax.lax.broadcasted_iota(jnp.int32, s.shape, 1)
    a = jnp.min(jnp.where(s == m, idx, _TILE))

    @pl.when(j == 0)
    def _():
        bestv_ref[0] = m
        besti_ref[0] = a

    @pl.when((j > 0) & (m > bestv_ref[0]))
    def _():
        bestv_ref[0] = m
        besti_ref[0] = j * _TILE + a

    @pl.when(j == nj - 1)
    def _():
        p_ref[...] = jnp.full((1, 128), besti_ref[0].astype(jnp.float32))


def _select_body(p_hbm, x_hbm, r_hbm, out_hbm, p_v, x_v, r_v, o_v):
    wid = lax.axis_index("s")
    row = wid // (2048 // _XPW)
    col = (wid % (2048 // _XPW)) * _XPW
    pltpu.sync_copy(p_hbm.at[0], p_v)
    pltpu.sync_copy(x_hbm.at[row, pl.ds(col, _XPW)], x_v)
    pltpu.sync_copy(r_hbm.at[row, pl.ds(col, _XPW)], r_v)
    pb = p_v[pl.ds(0, _L)]

    for k in range(_XPW // _L):
        xk = x_v[pl.ds(k * _L, _L)]
        rk = r_v[pl.ds(k * _L, _L)]
        cond = (xk == jnp.float32(_MASK_TOK)) & (rk < jnp.float32(_ALPHA))
        o_v[pl.ds(k * _L, _L)] = jnp.where(cond, pb, xk)
    pltpu.sync_copy(o_v, out_hbm.at[row, pl.ds(col, _XPW)])


def kernel(X, rand_vals, emb, W, b):
    b2 = b.reshape(1, _VOCAB)
    p_row = pl.pallas_call(
        _matvec_body,
        grid=(_VOCAB // _TILE,),
        in_specs=[
            pl.BlockSpec((8, _D), lambda j: (0, 0)),
            pl.BlockSpec((_D, _TILE), lambda j: (0, j)),
            pl.BlockSpec((1, _TILE), lambda j: (0, j)),
        ],
        out_specs=pl.BlockSpec((1, 128), lambda j: (0, 0)),
        out_shape=jax.ShapeDtypeStruct((1, 128), jnp.float32),
        scratch_shapes=[
            pltpu.SMEM((1,), jnp.float32),
            pltpu.SMEM((1,), jnp.int32),
        ],
    )(emb, W, b2)

    out = pl.kernel(
        _select_body,
        out_type=jax.ShapeDtypeStruct(X.shape, jnp.float32),
        mesh=_mesh,
        scratch_types=[
            pltpu.VMEM((128,), jnp.float32),
            pltpu.VMEM((_XPW,), jnp.float32),
            pltpu.VMEM((_XPW,), jnp.float32),
            pltpu.VMEM((_XPW,), jnp.float32),
        ],
    )(p_row, X, rand_vals)
    return out
